# Initial kernel scaffold; baseline (speedup 1.0000x reference)
import functools

import jax
import jax.numpy as jnp
from jax import lax
from jax.experimental import pallas as pl
from jax.experimental.pallas import tpu as pltpu
from jax.experimental.pallas import tpu_sc as plsc

F = 26
D = 16
B = 16384
FIELD = 40000
L = 16
NC, NS = 2, 16
NW = NC * NS
BPW = B // NW
CB = 128
NCHUNK = BPW // CB


def _fm_body(xtb, t1, t2, bias, out,
             xv, idxv, rowsv, t1rv, linv, resv, outv, biasv, sem1, sem2):
    wid = lax.axis_index("s") * NC + lax.axis_index("c")
    base = wid * BPW
    pltpu.sync_copy(bias, biasv)

    for g in range(NCHUNK):
        cbase = base + g * CB
        pltpu.sync_copy(xtb.at[wid * NCHUNK + g], xv)

        def build(f, carry):
            off = f * FIELD
            for j in range(CB // L):
                idxv[f, pl.ds(j * L, L)] = xv[f, pl.ds(j * L, L)] + off
            return carry
        lax.fori_loop(0, F, build, 0)

        cp2 = pltpu.async_copy(t2.at[idxv], rowsv, sem1)
        cp1 = pltpu.async_copy(t1.at[idxv], t1rv, sem2)

        cp1.wait()
        for j in range(CB // L):
            linv[pl.ds(j * L, L)] = jnp.zeros((L,), jnp.float32)

        def lin_acc(f, carry):
            for j in range(CB // L):
                linv[pl.ds(j * L, L)] = (linv[pl.ds(j * L, L)]
                                         + t1rv[f, pl.ds(j * L, L)])
            return carry
        lax.fori_loop(0, F, lin_acc, 0)

        cp2.wait()

        def per_b(c, carry):
            v0 = rowsv[0, c]
            acc = v0
            sq = v0 * v0
            for f in range(1, F):
                v = rowsv[f, c]
                acc = acc + v
                sq = sq + v * v
            r = acc * acc + sq
            resv[c] = jnp.sum(r)
            return carry
        lax.fori_loop(0, CB, per_b, 0)

        bval = biasv[0]
        for j in range(CB // L):
            t = linv[pl.ds(j * L, L)] + bval + resv[pl.ds(j * L, L)] * 0.5
            outv[pl.ds(j * L, L)] = 1.0 / (1.0 + jnp.exp(-t))
        pltpu.sync_copy(outv, out.at[pl.ds(cbase, CB)])


def _fm(xtb, t1, t2, bias):
    mesh = plsc.VectorSubcoreMesh(core_axis_name="c", subcore_axis_name="s")
    run = pl.kernel(
        _fm_body,
        out_type=jax.ShapeDtypeStruct((B,), jnp.float32),
        mesh=mesh,
        scratch_types=[
            pltpu.VMEM((F, CB), jnp.int32),
            pltpu.VMEM((F, CB), jnp.int32),
            pltpu.VMEM((F, CB, D), jnp.float32),
            pltpu.VMEM((F, CB), jnp.float32),
            pltpu.VMEM((CB,), jnp.float32),
            pltpu.VMEM((CB,), jnp.float32),
            pltpu.VMEM((CB,), jnp.float32),
            pltpu.VMEM((1,), jnp.float32),
            pltpu.SemaphoreType.DMA,
            pltpu.SemaphoreType.DMA,
        ],
    )
    return run(xtb, t1, t2, bias)


def kernel(x, table1, table2, bias):
    xtb = (x.astype(jnp.int32).T
           .reshape(F, NW * NCHUNK, CB)
           .transpose(1, 0, 2))
    t1 = table1.reshape(-1)
    out = _fm(xtb, t1, table2, bias)
    return out.reshape(B, 1)


# speedup vs baseline: 1.6985x; 1.6985x over previous
import functools

import jax
import jax.numpy as jnp
from jax import lax
from jax.experimental import pallas as pl
from jax.experimental.pallas import tpu as pltpu
from jax.experimental.pallas import tpu_sc as plsc

F = 26
D = 16
B = 16384
FIELD = 40000
TOTAL = F * FIELD
L = 16
NC, NS = 2, 16
NW = NC * NS
BPW = B // NW
CB = 128
NCHUNK = BPW // CB


def _fm_body(xtb, t1, t2, bias, out,
             xv, idxv, idx1v, rowsv, t1rv, linv, outv, biasv, sem1, sem2):
    wid = lax.axis_index("s") * NC + lax.axis_index("c")
    base = wid * BPW
    pltpu.sync_copy(bias, biasv.at[pl.ds(0, 1)])

    for g in range(NCHUNK):
        cbase = base + g * CB
        pltpu.sync_copy(xtb.at[wid * NCHUNK + g], xv)

        def build(f, carry):
            off = f * FIELD
            for j in range(CB // L):
                t = xv[f, pl.ds(j * L, L)] + off
                idx1v[pl.ds(f * CB + j * L, L)] = t
                p = (t & ~(CT - 1)) | ((t & (CQ - 1)) << 3) | ((t >> 10) & 7)
                idxv[pl.ds(f * CB + j * L, L)] = p
            return carry
        lax.fori_loop(0, F, build, 0)

        cp2 = pltpu.async_copy(t2.at[idxv], rowsv, sem1)
        cp1 = pltpu.async_copy(t1.at[idx1v], t1rv, sem2)

        cp1.wait()
        for j in range(CB // L):
            linv[pl.ds(j * L, L)] = jnp.zeros((L,), jnp.float32)

        def lin_acc(f, carry):
            for j in range(CB // L):
                linv[pl.ds(j * L, L)] = (linv[pl.ds(j * L, L)]
                                         + t1rv[pl.ds(f * CB + j * L, L)])
            return carry
        lax.fori_loop(0, F, lin_acc, 0)

        cp2.wait()

        bval = biasv[pl.ds(0, L)][0]
        lanes = lax.iota(jnp.int32, L)
        for j in range(CB // L):
            def per_b(i, cvec):
                c = j * L + i
                v0 = rowsv[c]
                acc = v0
                sq = v0 * v0
                for f in range(1, F):
                    v = rowsv[f * CB + c]
                    acc = acc + v
                    sq = sq + v * v
                r = acc * acc + sq
                s = jnp.sum(r)
                return jnp.where(lanes == i, s, cvec)
            resvec = lax.fori_loop(0, L, per_b,
                                   jnp.zeros((L,), jnp.float32))
            t = linv[pl.ds(j * L, L)] + bval + resvec * 0.5
            outv[pl.ds(j * L, L)] = 1.0 / (1.0 + jnp.exp(-t))
        pltpu.sync_copy(outv, out.at[pl.ds(cbase, CB)])


CT = 8192
CQ = CT // 8
NTB = -(-TOTAL // CT)
TPAD = NTB * CT


def _tr_body(in_ref, out_ref):
    t = in_ref[...]
    for s in range(8):
        out_ref[:, 16 * s:16 * (s + 1)] = t[:, s * CQ:(s + 1) * CQ].T


def _transpose_tc(t2t):
    return pl.pallas_call(
        _tr_body,
        grid=(NTB,),
        in_specs=[pl.BlockSpec((16, CT), lambda j: (0, j))],
        out_specs=pl.BlockSpec((CQ, 128), lambda j: (j, 0)),
        out_shape=jax.ShapeDtypeStruct((TPAD // 8, 128), jnp.float32),
    )(t2t)


def _fm(xtb, t1, t2, bias):
    mesh = plsc.VectorSubcoreMesh(core_axis_name="c", subcore_axis_name="s")
    run = pl.kernel(
        _fm_body,
        out_type=jax.ShapeDtypeStruct((B,), jnp.float32),
        mesh=mesh,
        compiler_params=pltpu.CompilerParams(
            needs_layout_passes=False, use_tc_tiling_on_sc=False),
        scratch_types=[
            pltpu.VMEM((F, CB), jnp.int32),
            pltpu.VMEM((F * CB,), jnp.int32),
            pltpu.VMEM((F * CB,), jnp.int32),
            pltpu.VMEM((F * CB, D), jnp.float32),
            pltpu.VMEM((F * CB,), jnp.float32),
            pltpu.VMEM((CB,), jnp.float32),
            pltpu.VMEM((CB,), jnp.float32),
            pltpu.VMEM((L,), jnp.float32),
            pltpu.SemaphoreType.DMA,
            pltpu.SemaphoreType.DMA,
        ],
    )
    return run(xtb, t1, t2, bias)


def kernel(x, table1, table2, bias):
    xtb = (x.astype(jnp.int32).T
           .reshape(F, NW * NCHUNK, CB)
           .transpose(1, 0, 2))
    t1 = table1.reshape(-1)
    t2r = _transpose_tc(table2.T).reshape(TPAD, D)
    out = _fm(xtb, t1, t2r, bias)
    return out.reshape(B, 1)
